# Initial kernel scaffold; baseline (speedup 1.0000x reference)
import jax
import jax.numpy as jnp
from jax.experimental import pallas as pl


def kernel(x, reg, Wg1, Wb1, Wc1, Wg2, Wb2, Wc2):
    raise NotImplementedError("write your pallas kernel here")


# speedup vs baseline: 6.0958x; 6.0958x over previous
import functools

import jax
import jax.numpy as jnp
from jax.experimental import pallas as pl
from jax.experimental.pallas import tpu as pltpu

_NEG_SLOPE = 0.2
_IN_EPS = 1e-5
_VMEM_LIMIT = 64 * 1024 * 1024


def _leaky(v):
    return jnp.where(v >= 0, v, _NEG_SLOPE * v)


def _fold_w3x3(w):
    Cout, Cin = w.shape[:2]
    return jnp.transpose(w, (0, 2, 3, 1)).reshape(Cout, 9 * Cin)


def _block_kernel(x_ref, reg_ref, wgb_ref, wc1_ref, wc2_ref, o_ref,
                  zbuf, slab, gbuf, *, C, H, W, base):
    P = H * W
    width = zbuf.shape[1]
    col = jax.lax.broadcasted_iota(jnp.int32, (1, P), 1) % W
    edge_l = col != 0
    edge_r = col != (W - 1)

    def build_slab(src):
        zbuf[:, 0:base] = jnp.zeros((C, base), jnp.float32)
        zbuf[:, base + P:] = jnp.zeros((C, width - base - P), jnp.float32)
        zbuf[:, base:base + P] = src
        for dy in range(3):
            for dx in range(3):
                t = dy * 3 + dx
                off = (dy - 1) * W + (dx - 1)
                v = zbuf[:, base + off:base + off + P]
                if dx == 0:
                    v = jnp.where(edge_l, v, 0.0)
                elif dx == 2:
                    v = jnp.where(edge_r, v, 0.0)
                slab[t * C:(t + 1) * C, :] = v.astype(slab.dtype)

    def conv(w):
        return jax.lax.dot_general(
            w, slab[...], (((1,), (0,)), ((), ())),
            preferred_element_type=jnp.float32)

    inv = 1.0 / P

    def csain(v, g, b):
        s = jnp.sum(v, axis=1, keepdims=True)
        s2 = jnp.sum(v * v, axis=1, keepdims=True)
        mean = s * inv
        var = jnp.maximum(s2 * inv - mean * mean, 0.0)
        xn = (v - mean) * jax.lax.rsqrt(var + _IN_EPS)
        return _leaky((1.0 + g) * xn + b)

    build_slab(reg_ref[0])
    for i in range(4):
        gbuf[i * C:(i + 1) * C, :] = _leaky(conv(wgb_ref[i * C:(i + 1) * C, :]))

    x = x_ref[0]
    t1 = csain(x, gbuf[0:C, :], gbuf[C:2 * C, :])
    build_slab(t1)
    h1 = conv(wc1_ref[...])
    t2 = csain(h1, gbuf[2 * C:3 * C, :], gbuf[3 * C:4 * C, :])
    build_slab(t2)
    o_ref[0] = conv(wc2_ref[...]) + x


def kernel(x, reg, Wg1, Wb1, Wc1, Wg2, Wb2, Wc2):
    N, C, H, W = x.shape
    P = H * W
    x_pp = x.reshape(N, C, P)
    reg_pp = reg.reshape(N, C, P)

    w_gb = jnp.concatenate(
        [_fold_w3x3(Wg1), _fold_w3x3(Wb1), _fold_w3x3(Wg2), _fold_w3x3(Wb2)],
        axis=0).astype(jnp.bfloat16)
    wc1 = _fold_w3x3(Wc1).astype(jnp.bfloat16)
    wc2 = _fold_w3x3(Wc2).astype(jnp.bfloat16)
    K9 = w_gb.shape[1]

    base = max(64, W + 1)
    width = -(-(2 * base + P) // 128) * 128

    body = functools.partial(_block_kernel, C=C, H=H, W=W, base=base)
    out = pl.pallas_call(
        body,
        out_shape=jax.ShapeDtypeStruct((N, C, P), jnp.float32),
        grid=(N,),
        in_specs=[
            pl.BlockSpec((1, C, P), lambda n: (n, 0, 0)),
            pl.BlockSpec((1, C, P), lambda n: (n, 0, 0)),
            pl.BlockSpec((4 * C, K9), lambda n: (0, 0)),
            pl.BlockSpec((C, K9), lambda n: (0, 0)),
            pl.BlockSpec((C, K9), lambda n: (0, 0)),
        ],
        out_specs=pl.BlockSpec((1, C, P), lambda n: (n, 0, 0)),
        scratch_shapes=[
            pltpu.VMEM((C, width), jnp.float32),
            pltpu.VMEM((9 * C, P), jnp.bfloat16),
            pltpu.VMEM((4 * C, P), jnp.float32),
        ],
        compiler_params=pltpu.CompilerParams(
            dimension_semantics=("parallel",),
            vmem_limit_bytes=_VMEM_LIMIT),
    )(x_pp, reg_pp, w_gb, wc1, wc2)
    return out.reshape(N, C, H, W)
